# baseline (speedup 1.0000x reference)
import functools

import jax
import jax.numpy as jnp
from jax import lax
from jax.experimental import pallas as pl
from jax.experimental.pallas import tpu as pltpu
from jax.experimental.pallas import tpu_sc as plsc

NUM_CLASS = 1000
N_CLS_CTX = 4
CTX_DIM = 512
PREFIX_LEN = 6
SUFFIX_LEN = 67
SEQ_LEN = PREFIX_LEN + N_CLS_CTX + SUFFIX_LEN
BATCH = 1024

NC = 2
NS = 16
NW = NC * NS
BPW = BATCH // NW

BB = 8


def _gather_body(cls_hbm, idx_hbm, out_hbm, idx_v, rows_v, gsem):
    wid = lax.axis_index("s") * NC + lax.axis_index("c")
    base = wid * BPW
    pltpu.sync_copy(idx_hbm.at[pl.ds(base, BPW)], idx_v)
    pltpu.async_copy(cls_hbm.at[idx_v], rows_v, gsem).wait()
    pltpu.sync_copy(rows_v, out_hbm.at[pl.ds(base, BPW)])


def _sc_gather(label, cls_ctx):
    mesh = plsc.VectorSubcoreMesh(core_axis_name="c", subcore_axis_name="s")
    return pl.kernel(
        _gather_body,
        out_type=jax.ShapeDtypeStruct((BATCH, N_CLS_CTX, CTX_DIM),
                                      jnp.float32),
        mesh=mesh,
        scratch_types=[
            pltpu.VMEM((BPW,), jnp.int32),
            pltpu.VMEM((BPW, N_CLS_CTX, CTX_DIM), jnp.float32),
            pltpu.SemaphoreType.DMA,
        ],
    )(cls_ctx, label)


def _assemble_body(pref_ref, suf_ref, cls_ref, out_ref):
    out_ref[:, 0:PREFIX_LEN, :] = jnp.broadcast_to(
        pref_ref[...][None], (BB, PREFIX_LEN, CTX_DIM))
    out_ref[:, PREFIX_LEN:PREFIX_LEN + N_CLS_CTX, :] = cls_ref[...]
    out_ref[:, PREFIX_LEN + N_CLS_CTX:, :] = jnp.broadcast_to(
        suf_ref[...][None], (BB, SUFFIX_LEN, CTX_DIM))


def _tc_assemble(pref, suf, rows):
    return pl.pallas_call(
        _assemble_body,
        grid=(BATCH // BB,),
        in_specs=[
            pl.BlockSpec((PREFIX_LEN, CTX_DIM), lambda i: (0, 0)),
            pl.BlockSpec((SUFFIX_LEN, CTX_DIM), lambda i: (0, 0)),
            pl.BlockSpec((BB, N_CLS_CTX, CTX_DIM), lambda i: (i, 0, 0)),
        ],
        out_specs=pl.BlockSpec((BB, SEQ_LEN, CTX_DIM), lambda i: (i, 0, 0)),
        out_shape=jax.ShapeDtypeStruct((BATCH, SEQ_LEN, CTX_DIM),
                                       jnp.float32),
        compiler_params=pltpu.CompilerParams(
            dimension_semantics=("arbitrary",)),
    )(pref, suf, rows)


@jax.jit
def _prompt_learner(label, cls_ctx, pref, suf):
    rows = _sc_gather(label, cls_ctx)
    return _tc_assemble(pref, suf, rows)


def kernel(label, cls_ctx, token_prefix, token_suffix):
    label = label.astype(jnp.int32)
    pref = token_prefix.reshape(PREFIX_LEN, CTX_DIM)
    suf = token_suffix.reshape(SUFFIX_LEN, CTX_DIM)
    return _prompt_learner(label, cls_ctx, pref, suf)


# speedup vs baseline: 2.2848x; 2.2848x over previous
import jax
import jax.numpy as jnp
from jax import lax
from jax.experimental import pallas as pl
from jax.experimental.pallas import tpu as pltpu
from jax.experimental.pallas import tpu_sc as plsc

NUM_CLASS = 1000
N_CLS_CTX = 4
CTX_DIM = 512
PREFIX_LEN = 6
SUFFIX_LEN = 67
SEQ_LEN = PREFIX_LEN + N_CLS_CTX + SUFFIX_LEN
BATCH = 1024
LANES = 16
NCOL = CTX_DIM // LANES

NC = 2
NS = 16
NW = NC * NS
BPW = BATCH // NW

N_BCAST = SEQ_LEN - N_CLS_CTX
HALF = BATCH // 2
N_UNIT = N_BCAST * 2
MAX_UNITS = -(-N_UNIT // NW) + 1
REP = 32
DMA_PER_UNIT = HALF // REP
CCH = 8


def _body(cls_hbm, idx_hbm, tmpl_hbm, out_hbm,
          idx_v, tmpl_v, rep0, rep1, rows_v, crep0, crep1,
          gsem, bsem0, bsem1, csem0, csem1):
    wid = lax.axis_index("s") * NC + lax.axis_index("c")
    base = wid * BPW

    pltpu.sync_copy(idx_hbm.at[pl.ds(base, BPW)], idx_v)
    pltpu.sync_copy(tmpl_hbm, tmpl_v)

    reps = [rep0, rep1]
    bsems = [bsem0, bsem1]

    start = (wid * N_UNIT) >> 5
    end = ((wid + 1) * N_UNIT) >> 5

    for k in range(MAX_UNITS):
        u = start + k

        @pl.when(u < end)
        def _do_unit(u=u, k=k):
            rep = reps[k % 2]
            sem = bsems[k % 2]
            if k >= 2:
                for _ in range(DMA_PER_UNIT):
                    pltpu.make_async_copy(
                        out_hbm.at[0, pl.ds(0, REP)], rep, sem).wait()
            ub = u >> 1
            s = jnp.where(ub >= PREFIX_LEN, ub + N_CLS_CTX, ub)
            half = u & 1

            def fill(row, carry):
                for c in range(NCOL):
                    rep[row, pl.ds(c * LANES, LANES)] = (
                        tmpl_v[s, pl.ds(c * LANES, LANES)])
                return carry
            lax.fori_loop(0, REP, fill, 0)

            for m in range(DMA_PER_UNIT):
                pltpu.async_copy(
                    rep, out_hbm.at[s, pl.ds(half * HALF + m * REP, REP)],
                    sem)

    creps = [crep0, crep1]
    csems = [csem0, csem1]
    for ch in range(BPW // CCH):
        pltpu.async_copy(
            cls_hbm.at[idx_v.at[pl.ds(ch * CCH, CCH)]], rows_v, gsem).wait()
        for r in range(N_CLS_CTX):
            crep = creps[r % 2]
            csem = csems[r % 2]
            if ch > 0 or r >= 2:
                pltpu.make_async_copy(
                    out_hbm.at[0, pl.ds(0, CCH)], crep, csem).wait()

            def tpose(j, carry):
                for c in range(NCOL):
                    crep[j, pl.ds(c * LANES, LANES)] = (
                        rows_v[j, r, pl.ds(c * LANES, LANES)])
                return carry
            lax.fori_loop(0, CCH, tpose, 0)

            pltpu.async_copy(
                crep,
                out_hbm.at[PREFIX_LEN + r, pl.ds(base + ch * CCH, CCH)],
                csem)

    for _ in range(DMA_PER_UNIT):
        pltpu.make_async_copy(out_hbm.at[0, pl.ds(0, REP)], rep0, bsem0).wait()
        pltpu.make_async_copy(out_hbm.at[0, pl.ds(0, REP)], rep1, bsem1).wait()
    pltpu.make_async_copy(out_hbm.at[0, pl.ds(0, CCH)], crep0, csem0).wait()
    pltpu.make_async_copy(out_hbm.at[0, pl.ds(0, CCH)], crep1, csem1).wait()


@jax.jit
def _prompt_learner(label, cls_ctx, tmpl):
    mesh = plsc.VectorSubcoreMesh(core_axis_name="c", subcore_axis_name="s")
    out_t = pl.kernel(
        _body,
        out_type=jax.ShapeDtypeStruct((SEQ_LEN, BATCH, CTX_DIM), jnp.float32),
        mesh=mesh,
        scratch_types=[
            pltpu.VMEM((BPW,), jnp.int32),
            pltpu.VMEM((SEQ_LEN, CTX_DIM), jnp.float32),
            pltpu.VMEM((REP, CTX_DIM), jnp.float32),
            pltpu.VMEM((REP, CTX_DIM), jnp.float32),
            pltpu.VMEM((CCH, N_CLS_CTX, CTX_DIM), jnp.float32),
            pltpu.VMEM((CCH, CTX_DIM), jnp.float32),
            pltpu.VMEM((CCH, CTX_DIM), jnp.float32),
            pltpu.SemaphoreType.DMA,
            pltpu.SemaphoreType.DMA,
            pltpu.SemaphoreType.DMA,
            pltpu.SemaphoreType.DMA,
            pltpu.SemaphoreType.DMA,
        ],
    )(cls_ctx, label, tmpl)
    return jnp.transpose(out_t, (1, 0, 2))


def kernel(label, cls_ctx, token_prefix, token_suffix):
    label = label.astype(jnp.int32)
    tmpl = jnp.concatenate(
        [token_prefix.reshape(PREFIX_LEN, CTX_DIM),
         jnp.zeros((N_CLS_CTX, CTX_DIM), jnp.float32),
         token_suffix.reshape(SUFFIX_LEN, CTX_DIM)], axis=0)
    return _prompt_learner(label, cls_ctx, tmpl)


# speedup vs baseline: 2.4977x; 1.0932x over previous
import jax
import jax.numpy as jnp
from jax import lax
from jax.experimental import pallas as pl
from jax.experimental.pallas import tpu as pltpu
from jax.experimental.pallas import tpu_sc as plsc

NUM_CLASS = 1000
N_CLS_CTX = 4
CTX_DIM = 512
PREFIX_LEN = 6
SUFFIX_LEN = 67
SEQ_LEN = PREFIX_LEN + N_CLS_CTX + SUFFIX_LEN
BATCH = 1024
LANES = 16
NCOL = CTX_DIM // LANES

NC = 2
NS = 16
NW = NC * NS
BPW = BATCH // NW

N_BCAST = SEQ_LEN - N_CLS_CTX
UPS = 16
N_UNIT = N_BCAST * UPS
UROWS = BATCH // UPS
REP = 16
DMA_PER_UNIT = UROWS // REP
SUF_OFF = 8
STAGE_ROWS = SUF_OFF + SUFFIX_LEN
CCH = 8


def _body(cls_hbm, idx_hbm, pref_hbm, suf_hbm, out_hbm,
          idx_v, stage_v, rep_all, rows_v, crep0, crep1,
          gsem, bsem, csem0, csem1):
    wid = lax.axis_index("s") * NC + lax.axis_index("c")
    base = wid * BPW

    pltpu.sync_copy(idx_hbm.at[pl.ds(base, BPW)], idx_v)
    pltpu.sync_copy(pref_hbm, stage_v.at[pl.ds(0, PREFIX_LEN)])
    pltpu.sync_copy(suf_hbm, stage_v.at[pl.ds(SUF_OFF, SUFFIX_LEN)])

    start = (wid * N_UNIT) >> 5
    end = ((wid + 1) * N_UNIT) >> 5
    s_first = start >> 4

    def unit_body(u, carry):
        ub = u >> 4
        bucket = ub - s_first
        sout = jnp.where(ub >= PREFIX_LEN, ub + N_CLS_CTX, ub)
        srcrow = jnp.where(ub >= PREFIX_LEN, ub + 2, ub)

        @pl.when((u == start) | ((u & (UPS - 1)) == 0))
        def _fill():
            def fr(row, c2):
                for c in range(NCOL):
                    rep_all[bucket, row, pl.ds(c * LANES, LANES)] = (
                        stage_v[srcrow, pl.ds(c * LANES, LANES)])
                return c2
            lax.fori_loop(0, REP, fr, 0)

        row0 = (u & (UPS - 1)) * UROWS
        for q in range(DMA_PER_UNIT):
            pltpu.async_copy(
                rep_all.at[bucket],
                out_hbm.at[sout, pl.ds(row0 + q * REP, REP)], bsem)
        return carry

    lax.fori_loop(start, end, unit_body, 0)

    creps = [crep0, crep1]
    csems = [csem0, csem1]
    for ch in range(BPW // CCH):
        pltpu.async_copy(
            cls_hbm.at[idx_v.at[pl.ds(ch * CCH, CCH)]], rows_v, gsem).wait()
        for r in range(N_CLS_CTX):
            crep = creps[r % 2]
            csem = csems[r % 2]
            if ch > 0 or r >= 2:
                pltpu.make_async_copy(
                    out_hbm.at[0, pl.ds(0, CCH)], crep, csem).wait()

            def tpose(j, carry):
                for c in range(NCOL):
                    crep[j, pl.ds(c * LANES, LANES)] = (
                        rows_v[j, r, pl.ds(c * LANES, LANES)])
                return carry
            lax.fori_loop(0, CCH, tpose, 0)

            pltpu.async_copy(
                crep,
                out_hbm.at[PREFIX_LEN + r, pl.ds(base + ch * CCH, CCH)],
                csem)

    def drain(i, carry):
        for q in range(DMA_PER_UNIT):
            pltpu.make_async_copy(
                out_hbm.at[0, pl.ds(0, REP)], rep_all.at[0], bsem).wait()
        return carry
    lax.fori_loop(0, end - start, drain, 0)
    pltpu.make_async_copy(out_hbm.at[0, pl.ds(0, CCH)], crep0, csem0).wait()
    pltpu.make_async_copy(out_hbm.at[0, pl.ds(0, CCH)], crep1, csem1).wait()


@jax.jit
def _prompt_learner(label, cls_ctx, pref, suf):
    mesh = plsc.VectorSubcoreMesh(core_axis_name="c", subcore_axis_name="s")
    out_t = pl.kernel(
        _body,
        out_type=jax.ShapeDtypeStruct((SEQ_LEN, BATCH, CTX_DIM), jnp.float32),
        mesh=mesh,
        scratch_types=[
            pltpu.VMEM((BPW,), jnp.int32),
            pltpu.VMEM((STAGE_ROWS, CTX_DIM), jnp.float32),
            pltpu.VMEM((4, REP, CTX_DIM), jnp.float32),
            pltpu.VMEM((CCH, N_CLS_CTX, CTX_DIM), jnp.float32),
            pltpu.VMEM((CCH, CTX_DIM), jnp.float32),
            pltpu.VMEM((CCH, CTX_DIM), jnp.float32),
            pltpu.SemaphoreType.DMA,
            pltpu.SemaphoreType.DMA,
            pltpu.SemaphoreType.DMA,
            pltpu.SemaphoreType.DMA,
        ],
    )(cls_ctx, label, pref, suf)
    return jnp.transpose(out_t, (1, 0, 2))


def kernel(label, cls_ctx, token_prefix, token_suffix):
    label = label.astype(jnp.int32)
    pref = token_prefix.reshape(PREFIX_LEN, CTX_DIM)
    suf = token_suffix.reshape(SUFFIX_LEN, CTX_DIM)
    return _prompt_learner(label, cls_ctx, pref, suf)


# speedup vs baseline: 2.5055x; 1.0031x over previous
import jax
import jax.numpy as jnp
from jax import lax
from jax.experimental import pallas as pl
from jax.experimental.pallas import tpu as pltpu
from jax.experimental.pallas import tpu_sc as plsc

NUM_CLASS = 1000
N_CLS_CTX = 4
CTX_DIM = 512
PREFIX_LEN = 6
SUFFIX_LEN = 67
SEQ_LEN = PREFIX_LEN + N_CLS_CTX + SUFFIX_LEN
BATCH = 1024
LANES = 16
NCOL = CTX_DIM // LANES

NC = 2
NS = 16
NW = NC * NS
BPW = BATCH // NW

N_BCAST = SEQ_LEN - N_CLS_CTX
UPS = 16
N_UNIT = N_BCAST * UPS
UROWS = BATCH // UPS
REP = 16
DMA_PER_UNIT = UROWS // REP
SUF_OFF = 8
STAGE_ROWS = SUF_OFF + SUFFIX_LEN
CCH = 8


def _body(cls_hbm, idx_hbm, pref_hbm, suf_hbm, out_hbm,
          idx_v, stage_v, rep_all, rows_v, crep0, crep1,
          gsem, bsem, csem0, csem1, ssem):
    wid = lax.axis_index("s") * NC + lax.axis_index("c")
    base = wid * BPW

    c1 = pltpu.async_copy(idx_hbm.at[pl.ds(base, BPW)], idx_v, ssem)
    c2 = pltpu.async_copy(
        pref_hbm, stage_v.at[pl.ds(0, PREFIX_LEN * CTX_DIM)], ssem)
    c3 = pltpu.async_copy(
        suf_hbm, stage_v.at[pl.ds(SUF_OFF * CTX_DIM, SUFFIX_LEN * CTX_DIM)],
        ssem)
    c1.wait(); c2.wait(); c3.wait()

    start = (wid * N_UNIT) >> 5
    end = ((wid + 1) * N_UNIT) >> 5
    s_first = start >> 4

    def unit_body(u, carry):
        ub = u >> 4
        bucket = ub - s_first
        sout = jnp.where(ub >= PREFIX_LEN, ub + N_CLS_CTX, ub)
        srcoff = jnp.where(ub >= PREFIX_LEN, ub + 2, ub) * CTX_DIM

        @pl.when((u == start) | ((u & (UPS - 1)) == 0))
        def _fill():
            vals = [stage_v[pl.ds(srcoff + c * LANES, LANES)]
                    for c in range(NCOL)]

            def fr(row, vs):
                for c in range(NCOL):
                    rep_all[bucket, row, pl.ds(c * LANES, LANES)] = vs[c]
                return vs
            lax.fori_loop(0, REP, fr, vals)

        row0 = (u & (UPS - 1)) * UROWS
        for q in range(DMA_PER_UNIT):
            pltpu.async_copy(
                rep_all.at[bucket],
                out_hbm.at[sout, pl.ds(row0 + q * REP, REP)], bsem)
        return carry

    lax.fori_loop(start, end, unit_body, 0)

    creps = [crep0, crep1]
    csems = [csem0, csem1]
    for ch in range(BPW // CCH):
        pltpu.async_copy(
            cls_hbm.at[idx_v.at[pl.ds(ch * CCH, CCH)]], rows_v, gsem).wait()
        for r in range(N_CLS_CTX):
            crep = creps[r % 2]
            csem = csems[r % 2]
            if ch > 0 or r >= 2:
                pltpu.make_async_copy(
                    out_hbm.at[0, pl.ds(0, CCH)], crep, csem).wait()

            def tpose(j, carry):
                for c in range(NCOL):
                    crep[j, pl.ds(c * LANES, LANES)] = (
                        rows_v[j, r, pl.ds(c * LANES, LANES)])
                return carry
            lax.fori_loop(0, CCH, tpose, 0)

            pltpu.async_copy(
                crep,
                out_hbm.at[PREFIX_LEN + r, pl.ds(base + ch * CCH, CCH)],
                csem)

    def drain(i, carry):
        for q in range(DMA_PER_UNIT):
            pltpu.make_async_copy(
                out_hbm.at[0, pl.ds(0, REP)], rep_all.at[0], bsem).wait()
        return carry
    lax.fori_loop(0, end - start, drain, 0)
    pltpu.make_async_copy(out_hbm.at[0, pl.ds(0, CCH)], crep0, csem0).wait()
    pltpu.make_async_copy(out_hbm.at[0, pl.ds(0, CCH)], crep1, csem1).wait()


@jax.jit
def _prompt_learner(label, cls_ctx, pref, suf):
    mesh = plsc.VectorSubcoreMesh(core_axis_name="c", subcore_axis_name="s")
    out_t = pl.kernel(
        _body,
        out_type=jax.ShapeDtypeStruct((SEQ_LEN, BATCH, CTX_DIM), jnp.float32),
        mesh=mesh,
        scratch_types=[
            pltpu.VMEM((BPW,), jnp.int32),
            pltpu.VMEM((STAGE_ROWS * CTX_DIM,), jnp.float32),
            pltpu.VMEM((4, REP, CTX_DIM), jnp.float32),
            pltpu.VMEM((CCH, N_CLS_CTX, CTX_DIM), jnp.float32),
            pltpu.VMEM((CCH, CTX_DIM), jnp.float32),
            pltpu.VMEM((CCH, CTX_DIM), jnp.float32),
            pltpu.SemaphoreType.DMA,
            pltpu.SemaphoreType.DMA,
            pltpu.SemaphoreType.DMA,
            pltpu.SemaphoreType.DMA,
            pltpu.SemaphoreType.DMA,
        ],
    )(cls_ctx, label, pref, suf)
    return jnp.transpose(out_t, (1, 0, 2))


def kernel(label, cls_ctx, token_prefix, token_suffix):
    label = label.astype(jnp.int32)
    pref = token_prefix.reshape(PREFIX_LEN * CTX_DIM)
    suf = token_suffix.reshape(SUFFIX_LEN * CTX_DIM)
    return _prompt_learner(label, cls_ctx, pref, suf)
